# Initial kernel scaffold; baseline (speedup 1.0000x reference)
import jax
import jax.numpy as jnp
from jax.experimental import pallas as pl


def kernel(x, edge_index, edge_type, batch, pi, W0, b0, Wh, bh, Wg, bg, g_gamma, g_beta, Wp, bp, p_gamma, p_beta, Wf, bf, f_gamma, f_beta):
    raise NotImplementedError("write your pallas kernel here")


# speedup vs baseline: 15.7720x; 15.7720x over previous
import functools

import jax
import jax.numpy as jnp
from jax import lax
from jax.experimental import pallas as pl
from jax.experimental.pallas import tpu as pltpu
from jax.experimental.pallas import tpu_sc as plsc

N = 10000
E = 320000
NDIM = 2
NL = 3
DH = 128
B = 64

NCORES = 2
NSUB = 16
NWORK = NCORES * NSUB
EPW = E // NWORK
G = 128
CHUNKS_CAP = 80
TRASH = N

NROWS_SC = 10112
RPT = NROWS_SC // NSUB

RB = 256
NPAD = 10240
NBLK = NPAD // RB

_MESH = dict(core_axis_name="c", subcore_axis_name="s",
             num_cores=NCORES, num_subcores=NSUB)


def _sc_prep(src, dst, et0, et1):
  mesh = plsc.VectorSubcoreMesh(**_MESH)
  out_type = (
      jax.ShapeDtypeStruct((NDIM, NWORK, CHUNKS_CAP, G), jnp.int32),
      jax.ShapeDtypeStruct((NDIM, NWORK, CHUNKS_CAP, G), jnp.int32),
      jax.ShapeDtypeStruct((NDIM, NWORK, 16), jnp.int32),
      jax.ShapeDtypeStruct((NDIM, NWORK, NPAD), jnp.float32),
  )
  scratch = [
      pltpu.VMEM((EPW,), jnp.int32),
      pltpu.VMEM((EPW,), jnp.int32),
      pltpu.VMEM((EPW,), jnp.int32),
      pltpu.VMEM((CHUNKS_CAP * G,), jnp.int32),
      pltpu.VMEM((CHUNKS_CAP * G,), jnp.int32),
      pltpu.VMEM((CHUNKS_CAP, G), jnp.int32),
      pltpu.VMEM((CHUNKS_CAP, G), jnp.int32),
      pltpu.VMEM((16,), jnp.int32),
      pltpu.VMEM((NPAD,), jnp.float32),
  ]

  @functools.partial(
      pl.kernel, out_type=out_type, mesh=mesh, scratch_types=scratch,
      compiler_params=pltpu.CompilerParams(needs_layout_passes=False),
  )
  def prep(src_h, dst_h, et0_h, et1_h, srcc_h, dstc_h, cnt_h, degp_h,
           se, de, ee, sflat, dflat, s2d, d2d, cnt_v, degl):
    cid = lax.axis_index("c")
    sid = lax.axis_index("s")
    wid = cid * NSUB + sid
    base = wid * EPW
    pltpu.sync_copy(src_h.at[pl.ds(base, EPW)], se)
    pltpu.sync_copy(dst_h.at[pl.ds(base, EPW)], de)

    ii = lax.iota(jnp.int32, 16)
    dummy = jnp.full((16,), TRASH, jnp.int32)
    fones = jnp.ones((16,), jnp.float32)
    z16 = jnp.zeros((16,), jnp.float32)

    for d, et_h in enumerate((et0_h, et1_h)):
      pltpu.sync_copy(et_h.at[pl.ds(base, EPW)], ee)

      @pl.loop(0, NPAD // 16)
      def _zdeg(i):
        degl[pl.ds(i * 16, 16)] = z16

      def cbody(i, c):
        sl = pl.ds(i * 16, 16)
        s16 = se[sl]
        d16 = de[sl]
        e16 = ee[sl]
        m = e16 == 1
        mi = m.astype(jnp.int32)
        pos = c + plsc.cumsum(mi) - 1
        plsc.store_scatter(sflat, [pos], s16, mask=m)
        plsc.store_scatter(dflat, [pos], d16, mask=m)
        plsc.addupdate_scatter(degl, [d16], fones, mask=m)
        return c + jnp.sum(mi)
      c = lax.fori_loop(0, EPW // 16, cbody, jnp.int32(0), unroll=2)

      for t in range(G // 16):
        idx = c + t * 16 + ii
        plsc.store_scatter(sflat, [idx], dummy)
        plsc.store_scatter(dflat, [idx], dummy)
      nch = (c + G - 1) // G

      def copybody(j, _):
        r = j // (G // 16)
        cc = (j % (G // 16)) * 16
        s2d[r, pl.ds(cc, 16)] = sflat[pl.ds(j * 16, 16)]
        d2d[r, pl.ds(cc, 16)] = dflat[pl.ds(j * 16, 16)]
        return 0
      lax.fori_loop(0, nch * (G // 16), copybody, 0)

      cnt_v[...] = jnp.full((16,), nch, jnp.int32)
      pltpu.sync_copy(cnt_v, cnt_h.at[d, wid])
      pltpu.sync_copy(s2d, srcc_h.at[d, wid])
      pltpu.sync_copy(d2d, dstc_h.at[d, wid])
      pltpu.sync_copy(degl, degp_h.at[d, wid])

  return prep(src, dst, et0, et1)


def _sc_scatter(d, y, srcc, dstc, counts, width=DH):
  mesh = plsc.VectorSubcoreMesh(**_MESH)
  out_type = jax.ShapeDtypeStruct((NCORES, NPAD, width), jnp.float32)
  scratch = [
      pltpu.VMEM((CHUNKS_CAP, G), jnp.int32),
      pltpu.VMEM((CHUNKS_CAP, G), jnp.int32),
      pltpu.VMEM((16,), jnp.int32),
      pltpu.VMEM((G, width), jnp.float32),
      pltpu.VMEM((64, width), jnp.float32),
      pltpu.VMEM_SHARED((NROWS_SC, width), jnp.float32),
      pltpu.SemaphoreType.DMA,
  ]

  @functools.partial(
      pl.kernel, out_type=out_type, mesh=mesh, scratch_types=scratch,
      compiler_params=pltpu.CompilerParams(needs_layout_passes=False),
  )
  def scat(y_h, srcc_h, dstc_h, cnt_h, acc_h, sv, dv, cv, rows, zrow,
           accsh, sem):
    cid = lax.axis_index("c")
    sid = lax.axis_index("s")
    wid = cid * NSUB + sid
    pltpu.sync_copy(cnt_h.at[d, wid], cv)
    pltpu.sync_copy(srcc_h.at[d, wid], sv)
    pltpu.sync_copy(dstc_h.at[d, wid], dv)

    z16 = jnp.zeros((16,), jnp.float32)

    @pl.loop(0, 64)
    def _zfill(i):
      for c8 in range(width // 16):
        zrow[i, pl.ds(c8 * 16, 16)] = z16

    r0 = sid * RPT
    for k in range(RPT // 64):
      pltpu.sync_copy(zrow, accsh.at[pl.ds(r0 + k * 64, 64)])
    rem = RPT % 64
    if rem:
      pltpu.sync_copy(zrow.at[pl.ds(0, rem)],
                      accsh.at[pl.ds(r0 + (RPT // 64) * 64, rem)])
    plsc.subcore_barrier()

    nch = cv[...][0]

    def chunk(j, _):
      pltpu.async_copy(y_h.at[sv.at[j]], rows, sem).wait()
      pltpu.sync_copy(rows, accsh.at[dv.at[j]], add=True)
      return 0
    lax.fori_loop(0, nch, chunk, 0)

    plsc.subcore_barrier()
    pltpu.sync_copy(accsh.at[pl.ds(r0, RPT)],
                    acc_h.at[cid, pl.ds(r0, RPT)])

  return scat(y, srcc, dstc, counts)


def _dinv_block(dg_ref):
  deg = 1.0 + dg_ref[...]
  return 1.0 / jnp.sqrt(deg)


def _tc_first(xp, W, degp_d):
  def body(x_ref, w_ref, dg_ref, y_ref):
    i = pl.program_id(0)
    rows = i * RB + lax.broadcasted_iota(jnp.int32, (RB, 1), 0)
    dinv = _dinv_block(dg_ref)
    xw = jnp.dot(x_ref[...], w_ref[...], preferred_element_type=jnp.float32)
    y_ref[...] = jnp.where(rows < N, dinv * xw, 0.0)

  return pl.pallas_call(
      body,
      grid=(NBLK,),
      in_specs=[
          pl.BlockSpec((RB, DH), lambda i: (i, 0)),
          pl.BlockSpec((DH, DH), lambda i: (0, 0)),
          pl.BlockSpec((RB, 1), lambda i: (i, 0)),
      ],
      out_specs=pl.BlockSpec((RB, DH), lambda i: (i, 0)),
      out_shape=jax.ShapeDtypeStruct((NPAD, DH), jnp.float32),
  )(xp, W, degp_d)


def _tc_mid(accp, y, degp_d, bias, Wn):
  def body(a_ref, y_ref, dg_ref, b_ref, w_ref, h_ref, y2_ref):
    i = pl.program_id(0)
    rows = i * RB + lax.broadcasted_iota(jnp.int32, (RB, 1), 0)
    dinv = _dinv_block(dg_ref)
    s = a_ref[0] + a_ref[1] + y_ref[...]
    h = jnp.maximum(dinv * s + b_ref[...], 0.0)
    h = jnp.where(rows < N, h, 0.0)
    h_ref[...] = h
    y2 = dinv * jnp.dot(h, w_ref[...], preferred_element_type=jnp.float32)
    y2_ref[...] = jnp.where(rows < N, y2, 0.0)

  return pl.pallas_call(
      body,
      grid=(NBLK,),
      in_specs=[
          pl.BlockSpec((NCORES, RB, DH), lambda i: (0, i, 0)),
          pl.BlockSpec((RB, DH), lambda i: (i, 0)),
          pl.BlockSpec((RB, 1), lambda i: (i, 0)),
          pl.BlockSpec((1, DH), lambda i: (0, 0)),
          pl.BlockSpec((DH, DH), lambda i: (0, 0)),
      ],
      out_specs=[
          pl.BlockSpec((RB, DH), lambda i: (i, 0)),
          pl.BlockSpec((RB, DH), lambda i: (i, 0)),
      ],
      out_shape=[
          jax.ShapeDtypeStruct((NPAD, DH), jnp.float32),
          jax.ShapeDtypeStruct((NPAD, DH), jnp.float32),
      ],
  )(accp, y, degp_d, bias, Wn)


def _tc_last(accp, y, degp_d, bias):
  def body(a_ref, y_ref, dg_ref, b_ref, h_ref):
    i = pl.program_id(0)
    rows = i * RB + lax.broadcasted_iota(jnp.int32, (RB, 1), 0)
    dinv = _dinv_block(dg_ref)
    s = a_ref[0] + a_ref[1] + y_ref[...]
    h = jnp.maximum(dinv * s + b_ref[...], 0.0)
    h_ref[...] = jnp.where(rows < N, h, 0.0)

  return pl.pallas_call(
      body,
      grid=(NBLK,),
      in_specs=[
          pl.BlockSpec((NCORES, RB, DH), lambda i: (0, i, 0)),
          pl.BlockSpec((RB, DH), lambda i: (i, 0)),
          pl.BlockSpec((RB, 1), lambda i: (i, 0)),
          pl.BlockSpec((1, DH), lambda i: (0, 0)),
      ],
      out_specs=pl.BlockSpec((RB, DH), lambda i: (i, 0)),
      out_shape=jax.ShapeDtypeStruct((NPAD, DH), jnp.float32),
  )(accp, y, degp_d, bias)


def _tc_pool(hs, batchp):
  njk = len(hs)

  def body(b_ref, *rest):
    h_refs = rest[:njk]
    out_ref = rest[njk]
    i = pl.program_id(0)

    @pl.when(i == 0)
    def _init():
      out_ref[...] = jnp.full((B, njk * DH), -jnp.inf, jnp.float32)

    rows = i * RB + lax.broadcasted_iota(jnp.int32, (RB, 1), 0)
    valid = rows < N
    bb = b_ref[...]
    hcat = jnp.concatenate([h[...] for h in h_refs], axis=1)
    b_lo = jnp.min(jnp.where(valid, bb, B - 1))
    b_hi = jnp.max(jnp.where(valid, bb, 0))

    def bbody(b, _):
      m = (bb == b) & valid
      mx = jnp.max(jnp.where(m, hcat, -jnp.inf), axis=0, keepdims=True)
      out_ref[pl.ds(b, 1), :] = jnp.maximum(out_ref[pl.ds(b, 1), :], mx)
      return 0
    lax.fori_loop(b_lo, b_hi + 1, bbody, 0)

  return pl.pallas_call(
      body,
      grid=(NBLK,),
      in_specs=[pl.BlockSpec((RB, 1), lambda i: (i, 0))] +
               [pl.BlockSpec((RB, DH), lambda i: (i, 0))] * njk,
      out_specs=pl.BlockSpec((B, njk * DH), lambda i: (0, 0)),
      out_shape=jax.ShapeDtypeStruct((B, njk * DH), jnp.float32),
  )(batchp, *hs)


def _tc_head(pooled, pi, Wg, bg, gg, gb, Wp, bp, pg, pb, Wf, bf, fg, fb):
  def ln(v, g, beta):
    mu = jnp.mean(v, axis=-1, keepdims=True)
    var = jnp.mean((v - mu) ** 2, axis=-1, keepdims=True)
    return (v - mu) / jnp.sqrt(var + 1e-5) * g + beta

  def body(pool_ref, pi_ref, wg_ref, bg_ref, gg_ref, gb_ref,
           wp_ref, bp_ref, pg_ref, pb_ref, wf_ref, bf_ref, fg_ref, fb_ref,
           out_ref):
    g = jnp.dot(pool_ref[...], wg_ref[...],
                preferred_element_type=jnp.float32) + bg_ref[...]
    g = jnp.maximum(ln(g, gg_ref[...], gb_ref[...]), 0.0)
    p = jnp.dot(pi_ref[...], wp_ref[...],
                preferred_element_type=jnp.float32) + bp_ref[...]
    p = jnp.maximum(ln(p, pg_ref[...], pb_ref[...]), 0.0)
    h = jnp.concatenate([g, p], axis=1)
    o = jnp.dot(h, wf_ref[...], preferred_element_type=jnp.float32) \
        + bf_ref[...]
    out_ref[...] = ln(o, fg_ref[...], fb_ref[...])

  args = (pooled, pi, Wg, bg, gg, gb, Wp, bp, pg, pb, Wf, bf, fg, fb)

  def _spec(a):
    zeros = tuple(0 for _ in a.shape)
    return pl.BlockSpec(a.shape, lambda z=zeros: z)

  return pl.pallas_call(
      body,
      in_specs=[_spec(a) for a in args],
      out_specs=pl.BlockSpec((B, Wf.shape[1]), lambda: (0, 0)),
      out_shape=jax.ShapeDtypeStruct((B, Wf.shape[1]), jnp.float32),
  )(*args)


def kernel(x, edge_index, edge_type, batch, pi, W0, b0, Wh, bh, Wg, bg,
           g_gamma, g_beta, Wp, bp, p_gamma, p_beta, Wf, bf, f_gamma,
           f_beta):
  src = edge_index[0].astype(jnp.int32)
  dst = edge_index[1].astype(jnp.int32)
  et0 = edge_type[:, 0].astype(jnp.int32)
  et1 = edge_type[:, 1].astype(jnp.int32)
  xp = jnp.pad(x, ((0, NPAD - N), (0, 0)))
  batchp = jnp.pad(batch.astype(jnp.int32), (0, NPAD - N)).reshape(NPAD, 1)

  srcc, dstc, counts, degp = _sc_prep(src, dst, et0, et1)

  hs = []
  for d in range(NDIM):
    dg = jnp.sum(degp[d], axis=0).reshape(NPAD, 1)
    y = _tc_first(xp, W0[d], dg)
    for l in range(NL):
      accp = _sc_scatter(d, y, srcc, dstc, counts)
      bias = (b0[d] if l == 0 else bh[d, l - 1]).reshape(1, DH)
      if l < NL - 1:
        h, y = _tc_mid(accp, y, dg, bias, Wh[d, l])
      else:
        h = _tc_last(accp, y, dg, bias)
      hs.append(h)

  pooled = _tc_pool(hs, batchp)
  out = _tc_head(pooled, pi,
                 Wg, bg.reshape(1, -1), g_gamma.reshape(1, -1),
                 g_beta.reshape(1, -1),
                 Wp, bp.reshape(1, -1), p_gamma.reshape(1, -1),
                 p_beta.reshape(1, -1),
                 Wf, bf.reshape(1, -1), f_gamma.reshape(1, -1),
                 f_beta.reshape(1, -1))
  return out


# speedup vs baseline: 18.6448x; 1.1822x over previous
import functools

import jax
import jax.numpy as jnp
from jax import lax
from jax.experimental import pallas as pl
from jax.experimental.pallas import tpu as pltpu
from jax.experimental.pallas import tpu_sc as plsc

N = 10000
E = 320000
NDIM = 2
NL = 3
DH = 128
B = 64

NCORES = 2
NSUB = 16
NWORK = NCORES * NSUB
EPW = E // NWORK
G = 128
CHUNKS_CAP = 80
TRASH = N

NROWS_SC = 10112
RPT = NROWS_SC // NSUB

RB = 256
NPAD = 10240
NBLK = NPAD // RB

_MESH = dict(core_axis_name="c", subcore_axis_name="s",
             num_cores=NCORES, num_subcores=NSUB)


def _sc_prep(src, dst, et0, et1):
  mesh = plsc.VectorSubcoreMesh(**_MESH)
  out_type = (
      jax.ShapeDtypeStruct((NDIM, NWORK, CHUNKS_CAP, G), jnp.int32),
      jax.ShapeDtypeStruct((NDIM, NWORK, CHUNKS_CAP, G), jnp.int32),
      jax.ShapeDtypeStruct((NDIM, NWORK, 16), jnp.int32),
      jax.ShapeDtypeStruct((NDIM, NWORK, NPAD), jnp.float32),
  )
  scratch = [
      pltpu.VMEM((EPW,), jnp.int32),
      pltpu.VMEM((EPW,), jnp.int32),
      pltpu.VMEM((EPW,), jnp.int32),
      pltpu.VMEM((CHUNKS_CAP * G,), jnp.int32),
      pltpu.VMEM((CHUNKS_CAP * G,), jnp.int32),
      pltpu.VMEM((CHUNKS_CAP, G), jnp.int32),
      pltpu.VMEM((CHUNKS_CAP, G), jnp.int32),
      pltpu.VMEM((16,), jnp.int32),
      pltpu.VMEM((NPAD,), jnp.float32),
  ]

  @functools.partial(
      pl.kernel, out_type=out_type, mesh=mesh, scratch_types=scratch,
      compiler_params=pltpu.CompilerParams(needs_layout_passes=False),
  )
  def prep(src_h, dst_h, et0_h, et1_h, srcc_h, dstc_h, cnt_h, degp_h,
           se, de, ee, sflat, dflat, s2d, d2d, cnt_v, degl):
    cid = lax.axis_index("c")
    sid = lax.axis_index("s")
    wid = cid * NSUB + sid
    base = wid * EPW
    pltpu.sync_copy(src_h.at[pl.ds(base, EPW)], se)
    pltpu.sync_copy(dst_h.at[pl.ds(base, EPW)], de)

    ii = lax.iota(jnp.int32, 16)
    dummy = jnp.full((16,), TRASH, jnp.int32)
    fones = jnp.ones((16,), jnp.float32)
    z16 = jnp.zeros((16,), jnp.float32)

    for d, et_h in enumerate((et0_h, et1_h)):
      pltpu.sync_copy(et_h.at[pl.ds(base, EPW)], ee)

      @pl.loop(0, NPAD // 16)
      def _zdeg(i):
        degl[pl.ds(i * 16, 16)] = z16

      def cbody(i, c):
        sl = pl.ds(i * 16, 16)
        s16 = se[sl]
        d16 = de[sl]
        e16 = ee[sl]
        m = e16 == 1
        mi = m.astype(jnp.int32)
        pos = c + plsc.cumsum(mi) - 1
        plsc.store_scatter(sflat, [pos], s16, mask=m)
        plsc.store_scatter(dflat, [pos], d16, mask=m)
        plsc.addupdate_scatter(degl, [d16], fones, mask=m)
        return c + jnp.sum(mi)
      c = lax.fori_loop(0, EPW // 16, cbody, jnp.int32(0), unroll=2)

      for t in range(G // 16):
        idx = c + t * 16 + ii
        plsc.store_scatter(sflat, [idx], dummy)
        plsc.store_scatter(dflat, [idx], dummy)
      nch = (c + G - 1) // G

      def copybody(j, _):
        r = j // (G // 16)
        cc = (j % (G // 16)) * 16
        s2d[r, pl.ds(cc, 16)] = sflat[pl.ds(j * 16, 16)]
        d2d[r, pl.ds(cc, 16)] = dflat[pl.ds(j * 16, 16)]
        return 0
      lax.fori_loop(0, nch * (G // 16), copybody, 0)

      cnt_v[...] = jnp.full((16,), nch, jnp.int32)
      pltpu.sync_copy(cnt_v, cnt_h.at[d, wid])
      pltpu.sync_copy(s2d, srcc_h.at[d, wid])
      pltpu.sync_copy(d2d, dstc_h.at[d, wid])
      pltpu.sync_copy(degl, degp_h.at[d, wid])

  return prep(src, dst, et0, et1)


_NROW = 2
_NIDX = 4


def _sc_scatter(d, y, srcc, dstc, counts, width=DH):
  mesh = plsc.VectorSubcoreMesh(**_MESH)
  out_type = jax.ShapeDtypeStruct((NCORES, NPAD, width), jnp.float32)
  scratch = [
      pltpu.VMEM((_NIDX, G), jnp.int32),
      pltpu.VMEM((_NIDX, G), jnp.int32),
      pltpu.VMEM((16,), jnp.int32),
      pltpu.VMEM((32, width), jnp.float32),
      pltpu.VMEM_SHARED((NROWS_SC, width), jnp.float32),
  ] + [pltpu.VMEM((G, width), jnp.float32) for _ in range(_NROW)] \
    + [pltpu.SemaphoreType.DMA for _ in range(2 * _NROW + _NIDX)]

  @functools.partial(
      pl.kernel, out_type=out_type, mesh=mesh, scratch_types=scratch,
      compiler_params=pltpu.CompilerParams(needs_layout_passes=False),
  )
  def scat(y_h, srcc_h, dstc_h, cnt_h, acc_h, sidx, didx, cv, zrow, accsh,
           *bufsem):
    rows = bufsem[:_NROW]
    gsem = bufsem[_NROW:2 * _NROW]
    ssem = bufsem[2 * _NROW:3 * _NROW]
    isem = bufsem[3 * _NROW:]
    cid = lax.axis_index("c")
    sid = lax.axis_index("s")
    wid = cid * NSUB + sid
    pltpu.sync_copy(cnt_h.at[d, wid], cv)

    def start_idx(j, b4):
      pltpu.async_copy(srcc_h.at[d, wid, j], sidx.at[b4], isem[b4])
      pltpu.async_copy(dstc_h.at[d, wid, j], didx.at[b4], isem[b4])

    def wait_idx(j, b4):
      pltpu.make_async_copy(srcc_h.at[d, wid, j], sidx.at[b4],
                            isem[b4]).wait()
      pltpu.make_async_copy(dstc_h.at[d, wid, j], didx.at[b4],
                            isem[b4]).wait()

    def start_gather(j, b, b4):
      pltpu.async_copy(y_h.at[sidx.at[b4]], rows[b], gsem[b])

    def wait_gather(b, b4):
      pltpu.make_async_copy(y_h.at[sidx.at[b4]], rows[b], gsem[b]).wait()

    def start_scatter(b, b4):
      pltpu.async_copy(rows[b], accsh.at[didx.at[b4]], ssem[b], add=True)

    def wait_scatter(b, b4):
      pltpu.make_async_copy(rows[b], accsh.at[didx.at[b4]], ssem[b]).wait()

    z16 = jnp.zeros((16,), jnp.float32)

    @pl.loop(0, 32)
    def _zfill(i):
      for c8 in range(width // 16):
        zrow[i, pl.ds(c8 * 16, 16)] = z16

    r0 = sid * RPT
    for k in range(RPT // 32):
      pltpu.sync_copy(zrow, accsh.at[pl.ds(r0 + k * 32, 32)])
    rem = RPT % 32
    if rem:
      pltpu.sync_copy(zrow.at[pl.ds(0, rem)],
                      accsh.at[pl.ds(r0 + (RPT // 32) * 32, rem)])
    plsc.subcore_barrier()

    nch = cv[...][0]

    for b in range(_NIDX):
      @pl.when(b < nch)
      def _pi(b=b):
        start_idx(b, b)
    for b in range(_NROW):
      @pl.when(b < nch)
      def _pg(b=b):
        wait_idx(b, b)
        start_gather(b, b, b)

    def outer(k, _):
      for u in range(_NIDX):
        j = k * _NIDX + u
        b = u % _NROW
        b4 = u

        @pl.when(j < nch)
        def _work(b=b, j=j, b4=b4):
          wait_gather(b, b4)
          start_scatter(b, b4)

        @pl.when(j + _NROW < nch)
        def _next(b=b, j=j, b4=b4):
          wait_scatter(b, b4)
          @pl.when(j + _NIDX < nch)
          def _refill(j=j, b4=b4):
            start_idx(j + _NIDX, b4)
          bn = (b4 + _NROW) % _NIDX
          wait_idx(j + _NROW, bn)
          start_gather(j + _NROW, b, bn)
      return 0
    lax.fori_loop(0, (nch + _NIDX - 1) // _NIDX, outer, 0)
    for b in range(_NROW):
      @pl.when(b < nch)
      def _drain(b=b):
        pltpu.make_async_copy(rows[b], accsh.at[didx.at[0]], ssem[b]).wait()

    plsc.subcore_barrier()
    pltpu.sync_copy(accsh.at[pl.ds(r0, RPT)],
                    acc_h.at[cid, pl.ds(r0, RPT)])

  return scat(y, srcc, dstc, counts)


def _dinv_block(dg_ref):
  deg = 1.0 + dg_ref[...]
  return 1.0 / jnp.sqrt(deg)


def _tc_first(xp, W, degp_d):
  def body(x_ref, w_ref, dg_ref, y_ref):
    i = pl.program_id(0)
    rows = i * RB + lax.broadcasted_iota(jnp.int32, (RB, 1), 0)
    dinv = _dinv_block(dg_ref)
    xw = jnp.dot(x_ref[...], w_ref[...], preferred_element_type=jnp.float32)
    y_ref[...] = jnp.where(rows < N, dinv * xw, 0.0)

  return pl.pallas_call(
      body,
      grid=(NBLK,),
      in_specs=[
          pl.BlockSpec((RB, DH), lambda i: (i, 0)),
          pl.BlockSpec((DH, DH), lambda i: (0, 0)),
          pl.BlockSpec((RB, 1), lambda i: (i, 0)),
      ],
      out_specs=pl.BlockSpec((RB, DH), lambda i: (i, 0)),
      out_shape=jax.ShapeDtypeStruct((NPAD, DH), jnp.float32),
  )(xp, W, degp_d)


def _tc_mid(accp, y, degp_d, bias, Wn):
  def body(a_ref, y_ref, dg_ref, b_ref, w_ref, h_ref, y2_ref):
    i = pl.program_id(0)
    rows = i * RB + lax.broadcasted_iota(jnp.int32, (RB, 1), 0)
    dinv = _dinv_block(dg_ref)
    s = a_ref[0] + a_ref[1] + y_ref[...]
    h = jnp.maximum(dinv * s + b_ref[...], 0.0)
    h = jnp.where(rows < N, h, 0.0)
    h_ref[...] = h
    y2 = dinv * jnp.dot(h, w_ref[...], preferred_element_type=jnp.float32)
    y2_ref[...] = jnp.where(rows < N, y2, 0.0)

  return pl.pallas_call(
      body,
      grid=(NBLK,),
      in_specs=[
          pl.BlockSpec((NCORES, RB, DH), lambda i: (0, i, 0)),
          pl.BlockSpec((RB, DH), lambda i: (i, 0)),
          pl.BlockSpec((RB, 1), lambda i: (i, 0)),
          pl.BlockSpec((1, DH), lambda i: (0, 0)),
          pl.BlockSpec((DH, DH), lambda i: (0, 0)),
      ],
      out_specs=[
          pl.BlockSpec((RB, DH), lambda i: (i, 0)),
          pl.BlockSpec((RB, DH), lambda i: (i, 0)),
      ],
      out_shape=[
          jax.ShapeDtypeStruct((NPAD, DH), jnp.float32),
          jax.ShapeDtypeStruct((NPAD, DH), jnp.float32),
      ],
  )(accp, y, degp_d, bias, Wn)


def _tc_last(accp, y, degp_d, bias):
  def body(a_ref, y_ref, dg_ref, b_ref, h_ref):
    i = pl.program_id(0)
    rows = i * RB + lax.broadcasted_iota(jnp.int32, (RB, 1), 0)
    dinv = _dinv_block(dg_ref)
    s = a_ref[0] + a_ref[1] + y_ref[...]
    h = jnp.maximum(dinv * s + b_ref[...], 0.0)
    h_ref[...] = jnp.where(rows < N, h, 0.0)

  return pl.pallas_call(
      body,
      grid=(NBLK,),
      in_specs=[
          pl.BlockSpec((NCORES, RB, DH), lambda i: (0, i, 0)),
          pl.BlockSpec((RB, DH), lambda i: (i, 0)),
          pl.BlockSpec((RB, 1), lambda i: (i, 0)),
          pl.BlockSpec((1, DH), lambda i: (0, 0)),
      ],
      out_specs=pl.BlockSpec((RB, DH), lambda i: (i, 0)),
      out_shape=jax.ShapeDtypeStruct((NPAD, DH), jnp.float32),
  )(accp, y, degp_d, bias)


def _tc_pool(hs, batchp):
  njk = len(hs)

  def body(b_ref, *rest):
    h_refs = rest[:njk]
    out_ref = rest[njk]
    i = pl.program_id(0)

    @pl.when(i == 0)
    def _init():
      out_ref[...] = jnp.full((B, njk * DH), -jnp.inf, jnp.float32)

    rows = i * RB + lax.broadcasted_iota(jnp.int32, (RB, 1), 0)
    valid = rows < N
    bb = b_ref[...]
    hcat = jnp.concatenate([h[...] for h in h_refs], axis=1)
    b_lo = jnp.min(jnp.where(valid, bb, B - 1))
    b_hi = jnp.max(jnp.where(valid, bb, 0))

    def bbody(b, _):
      m = (bb == b) & valid
      mx = jnp.max(jnp.where(m, hcat, -jnp.inf), axis=0, keepdims=True)
      out_ref[pl.ds(b, 1), :] = jnp.maximum(out_ref[pl.ds(b, 1), :], mx)
      return 0
    lax.fori_loop(b_lo, b_hi + 1, bbody, 0)

  return pl.pallas_call(
      body,
      grid=(NBLK,),
      in_specs=[pl.BlockSpec((RB, 1), lambda i: (i, 0))] +
               [pl.BlockSpec((RB, DH), lambda i: (i, 0))] * njk,
      out_specs=pl.BlockSpec((B, njk * DH), lambda i: (0, 0)),
      out_shape=jax.ShapeDtypeStruct((B, njk * DH), jnp.float32),
  )(batchp, *hs)


def _tc_head(pooled, pi, Wg, bg, gg, gb, Wp, bp, pg, pb, Wf, bf, fg, fb):
  def ln(v, g, beta):
    mu = jnp.mean(v, axis=-1, keepdims=True)
    var = jnp.mean((v - mu) ** 2, axis=-1, keepdims=True)
    return (v - mu) / jnp.sqrt(var + 1e-5) * g + beta

  def body(pool_ref, pi_ref, wg_ref, bg_ref, gg_ref, gb_ref,
           wp_ref, bp_ref, pg_ref, pb_ref, wf_ref, bf_ref, fg_ref, fb_ref,
           out_ref):
    g = jnp.dot(pool_ref[...], wg_ref[...],
                preferred_element_type=jnp.float32) + bg_ref[...]
    g = jnp.maximum(ln(g, gg_ref[...], gb_ref[...]), 0.0)
    p = jnp.dot(pi_ref[...], wp_ref[...],
                preferred_element_type=jnp.float32) + bp_ref[...]
    p = jnp.maximum(ln(p, pg_ref[...], pb_ref[...]), 0.0)
    h = jnp.concatenate([g, p], axis=1)
    o = jnp.dot(h, wf_ref[...], preferred_element_type=jnp.float32) \
        + bf_ref[...]
    out_ref[...] = ln(o, fg_ref[...], fb_ref[...])

  args = (pooled, pi, Wg, bg, gg, gb, Wp, bp, pg, pb, Wf, bf, fg, fb)

  def _spec(a):
    zeros = tuple(0 for _ in a.shape)
    return pl.BlockSpec(a.shape, lambda z=zeros: z)

  return pl.pallas_call(
      body,
      in_specs=[_spec(a) for a in args],
      out_specs=pl.BlockSpec((B, Wf.shape[1]), lambda: (0, 0)),
      out_shape=jax.ShapeDtypeStruct((B, Wf.shape[1]), jnp.float32),
  )(*args)


def kernel(x, edge_index, edge_type, batch, pi, W0, b0, Wh, bh, Wg, bg,
           g_gamma, g_beta, Wp, bp, p_gamma, p_beta, Wf, bf, f_gamma,
           f_beta):
  src = edge_index[0].astype(jnp.int32)
  dst = edge_index[1].astype(jnp.int32)
  et0 = edge_type[:, 0].astype(jnp.int32)
  et1 = edge_type[:, 1].astype(jnp.int32)
  xp = jnp.pad(x, ((0, NPAD - N), (0, 0)))
  batchp = jnp.pad(batch.astype(jnp.int32), (0, NPAD - N)).reshape(NPAD, 1)

  srcc, dstc, counts, degp = _sc_prep(src, dst, et0, et1)

  hs = []
  for d in range(NDIM):
    dg = jnp.sum(degp[d], axis=0).reshape(NPAD, 1)
    y = _tc_first(xp, W0[d], dg)
    for l in range(NL):
      accp = _sc_scatter(d, y, srcc, dstc, counts)
      bias = (b0[d] if l == 0 else bh[d, l - 1]).reshape(1, DH)
      if l < NL - 1:
        h, y = _tc_mid(accp, y, dg, bias, Wh[d, l])
      else:
        h = _tc_last(accp, y, dg, bias)
      hs.append(h)

  pooled = _tc_pool(hs, batchp)
  out = _tc_head(pooled, pi,
                 Wg, bg.reshape(1, -1), g_gamma.reshape(1, -1),
                 g_beta.reshape(1, -1),
                 Wp, bp.reshape(1, -1), p_gamma.reshape(1, -1),
                 p_beta.reshape(1, -1),
                 Wf, bf.reshape(1, -1), f_gamma.reshape(1, -1),
                 f_beta.reshape(1, -1))
  return out


# speedup vs baseline: 25.3469x; 1.3595x over previous
import functools

import jax
import jax.numpy as jnp
from jax import lax
from jax.experimental import pallas as pl
from jax.experimental.pallas import tpu as pltpu
from jax.experimental.pallas import tpu_sc as plsc

N = 10000
E = 320000
NDIM = 2
NL = 3
DH = 128
B = 64

NCORES = 2
NSUB = 16
NWORK = NCORES * NSUB
EPW = E // NWORK
G = 64
CHUNKS_CAP = 160
TRASH = N

NROWS_SC = 10112
RPT = NROWS_SC // NSUB

RB = 256
NPAD = 10240
NBLK = NPAD // RB

_MESH = dict(core_axis_name="c", subcore_axis_name="s",
             num_cores=NCORES, num_subcores=NSUB)


def _sc_prep(src, dst, et0, et1):
  mesh = plsc.VectorSubcoreMesh(**_MESH)
  out_type = (
      jax.ShapeDtypeStruct((NDIM, NWORK, CHUNKS_CAP, G), jnp.int32),
      jax.ShapeDtypeStruct((NDIM, NWORK, CHUNKS_CAP, G), jnp.int32),
      jax.ShapeDtypeStruct((NDIM, NWORK, 16), jnp.int32),
      jax.ShapeDtypeStruct((NDIM, NWORK, NPAD), jnp.float32),
  )
  scratch = [
      pltpu.VMEM((EPW,), jnp.int32),
      pltpu.VMEM((EPW,), jnp.int32),
      pltpu.VMEM((EPW,), jnp.int32),
      pltpu.VMEM((CHUNKS_CAP * G,), jnp.int32),
      pltpu.VMEM((CHUNKS_CAP * G,), jnp.int32),
      pltpu.VMEM((CHUNKS_CAP, G), jnp.int32),
      pltpu.VMEM((CHUNKS_CAP, G), jnp.int32),
      pltpu.VMEM((16,), jnp.int32),
      pltpu.VMEM((NPAD,), jnp.float32),
  ]

  @functools.partial(
      pl.kernel, out_type=out_type, mesh=mesh, scratch_types=scratch,
      compiler_params=pltpu.CompilerParams(needs_layout_passes=False),
  )
  def prep(src_h, dst_h, et0_h, et1_h, srcc_h, dstc_h, cnt_h, degp_h,
           se, de, ee, sflat, dflat, s2d, d2d, cnt_v, degl):
    cid = lax.axis_index("c")
    sid = lax.axis_index("s")
    wid = cid * NSUB + sid
    base = wid * EPW
    pltpu.sync_copy(src_h.at[pl.ds(base, EPW)], se)
    pltpu.sync_copy(dst_h.at[pl.ds(base, EPW)], de)

    ii = lax.iota(jnp.int32, 16)
    dummy = jnp.full((16,), TRASH, jnp.int32)
    fones = jnp.ones((16,), jnp.float32)
    z16 = jnp.zeros((16,), jnp.float32)

    for d, et_h in enumerate((et0_h, et1_h)):
      pltpu.sync_copy(et_h.at[pl.ds(base, EPW)], ee)

      @pl.loop(0, NPAD // 16)
      def _zdeg(i):
        degl[pl.ds(i * 16, 16)] = z16

      def cbody(i, c):
        sl = pl.ds(i * 16, 16)
        s16 = se[sl]
        d16 = de[sl]
        e16 = ee[sl]
        m = e16 == 1
        mi = m.astype(jnp.int32)
        pos = c + plsc.cumsum(mi) - 1
        plsc.store_scatter(sflat, [pos], s16, mask=m)
        plsc.store_scatter(dflat, [pos], d16, mask=m)
        plsc.addupdate_scatter(degl, [d16], fones, mask=m)
        return c + jnp.sum(mi)
      c = lax.fori_loop(0, EPW // 16, cbody, jnp.int32(0), unroll=2)

      for t in range(G // 16):
        idx = c + t * 16 + ii
        plsc.store_scatter(sflat, [idx], dummy)
        plsc.store_scatter(dflat, [idx], dummy)
      nch = (c + G - 1) // G

      def copybody(j, _):
        r = j // (G // 16)
        cc = (j % (G // 16)) * 16
        s2d[r, pl.ds(cc, 16)] = sflat[pl.ds(j * 16, 16)]
        d2d[r, pl.ds(cc, 16)] = dflat[pl.ds(j * 16, 16)]
        return 0
      lax.fori_loop(0, nch * (G // 16), copybody, 0)

      cnt_v[...] = jnp.full((16,), nch, jnp.int32)
      pltpu.sync_copy(cnt_v, cnt_h.at[d, wid])
      pltpu.sync_copy(s2d, srcc_h.at[d, wid])
      pltpu.sync_copy(d2d, dstc_h.at[d, wid])
      pltpu.sync_copy(degl, degp_h.at[d, wid])

  return prep(src, dst, et0, et1)


_NROW = 4
_NIDX = 8


def _sc_scatter(d, y, srcc, dstc, counts, width=DH):
  mesh = plsc.VectorSubcoreMesh(**_MESH)
  out_type = jax.ShapeDtypeStruct((NCORES, NPAD, width), jnp.float32)
  scratch = [
      pltpu.VMEM((_NIDX, G), jnp.int32),
      pltpu.VMEM((_NIDX, G), jnp.int32),
      pltpu.VMEM((16,), jnp.int32),
      pltpu.VMEM((32, width), jnp.float32),
      pltpu.VMEM_SHARED((NROWS_SC, width), jnp.float32),
  ] + [pltpu.VMEM((G, width), jnp.float32) for _ in range(_NROW)] \
    + [pltpu.SemaphoreType.DMA for _ in range(2 * _NROW + _NIDX)]

  @functools.partial(
      pl.kernel, out_type=out_type, mesh=mesh, scratch_types=scratch,
      compiler_params=pltpu.CompilerParams(needs_layout_passes=False),
  )
  def scat(y_h, srcc_h, dstc_h, cnt_h, acc_h, sidx, didx, cv, zrow, accsh,
           *bufsem):
    rows = bufsem[:_NROW]
    gsem = bufsem[_NROW:2 * _NROW]
    ssem = bufsem[2 * _NROW:3 * _NROW]
    isem = bufsem[3 * _NROW:]
    cid = lax.axis_index("c")
    sid = lax.axis_index("s")
    wid = cid * NSUB + sid
    pltpu.sync_copy(cnt_h.at[d, wid], cv)

    def start_idx(j, b4):
      pltpu.async_copy(srcc_h.at[d, wid, j], sidx.at[b4], isem[b4])
      pltpu.async_copy(dstc_h.at[d, wid, j], didx.at[b4], isem[b4])

    def wait_idx(j, b4):
      pltpu.make_async_copy(srcc_h.at[d, wid, j], sidx.at[b4],
                            isem[b4]).wait()
      pltpu.make_async_copy(dstc_h.at[d, wid, j], didx.at[b4],
                            isem[b4]).wait()

    def start_gather(j, b, b4):
      pltpu.async_copy(y_h.at[sidx.at[b4]], rows[b], gsem[b])

    def wait_gather(b, b4):
      pltpu.make_async_copy(y_h.at[sidx.at[b4]], rows[b], gsem[b]).wait()

    def start_scatter(b, b4):
      pltpu.async_copy(rows[b], accsh.at[didx.at[b4]], ssem[b], add=True)

    def wait_scatter(b, b4):
      pltpu.make_async_copy(rows[b], accsh.at[didx.at[b4]], ssem[b]).wait()

    z16 = jnp.zeros((16,), jnp.float32)

    @pl.loop(0, 32)
    def _zfill(i):
      for c8 in range(width // 16):
        zrow[i, pl.ds(c8 * 16, 16)] = z16

    r0 = sid * RPT
    for k in range(RPT // 32):
      pltpu.sync_copy(zrow, accsh.at[pl.ds(r0 + k * 32, 32)])
    rem = RPT % 32
    if rem:
      pltpu.sync_copy(zrow.at[pl.ds(0, rem)],
                      accsh.at[pl.ds(r0 + (RPT // 32) * 32, rem)])
    plsc.subcore_barrier()

    nch = cv[...][0]

    for b in range(_NIDX):
      @pl.when(b < nch)
      def _pi(b=b):
        start_idx(b, b)
    for b in range(_NROW):
      @pl.when(b < nch)
      def _pg(b=b):
        wait_idx(b, b)
        start_gather(b, b, b)

    def outer(k, _):
      for u in range(_NIDX):
        j = k * _NIDX + u
        b = u % _NROW
        b4 = u

        @pl.when(j < nch)
        def _work(b=b, j=j, b4=b4):
          wait_gather(b, b4)
          start_scatter(b, b4)

        @pl.when(j + _NROW < nch)
        def _next(b=b, j=j, b4=b4):
          wait_scatter(b, b4)
          @pl.when(j + _NIDX < nch)
          def _refill(j=j, b4=b4):
            start_idx(j + _NIDX, b4)
          bn = (b4 + _NROW) % _NIDX
          wait_idx(j + _NROW, bn)
          start_gather(j + _NROW, b, bn)
      return 0
    lax.fori_loop(0, (nch + _NIDX - 1) // _NIDX, outer, 0)
    for b in range(_NROW):
      @pl.when(b < nch)
      def _drain(b=b):
        pltpu.make_async_copy(rows[b], accsh.at[didx.at[0]], ssem[b]).wait()

    plsc.subcore_barrier()
    pltpu.sync_copy(accsh.at[pl.ds(r0, RPT)],
                    acc_h.at[cid, pl.ds(r0, RPT)])

  return scat(y, srcc, dstc, counts)


def _dinv_block(dg_ref):
  deg = 1.0 + dg_ref[...]
  return 1.0 / jnp.sqrt(deg)


def _tc_first(xp, W, degp_d):
  def body(x_ref, w_ref, dg_ref, y_ref):
    i = pl.program_id(0)
    rows = i * RB + lax.broadcasted_iota(jnp.int32, (RB, 1), 0)
    dinv = _dinv_block(dg_ref)
    xw = jnp.dot(x_ref[...], w_ref[...], preferred_element_type=jnp.float32)
    y_ref[...] = jnp.where(rows < N, dinv * xw, 0.0)

  return pl.pallas_call(
      body,
      grid=(NBLK,),
      in_specs=[
          pl.BlockSpec((RB, DH), lambda i: (i, 0)),
          pl.BlockSpec((DH, DH), lambda i: (0, 0)),
          pl.BlockSpec((RB, 1), lambda i: (i, 0)),
      ],
      out_specs=pl.BlockSpec((RB, DH), lambda i: (i, 0)),
      out_shape=jax.ShapeDtypeStruct((NPAD, DH), jnp.float32),
  )(xp, W, degp_d)


def _tc_mid(accp, y, degp_d, bias, Wn):
  def body(a_ref, y_ref, dg_ref, b_ref, w_ref, h_ref, y2_ref):
    i = pl.program_id(0)
    rows = i * RB + lax.broadcasted_iota(jnp.int32, (RB, 1), 0)
    dinv = _dinv_block(dg_ref)
    s = a_ref[0] + a_ref[1] + y_ref[...]
    h = jnp.maximum(dinv * s + b_ref[...], 0.0)
    h = jnp.where(rows < N, h, 0.0)
    h_ref[...] = h
    y2 = dinv * jnp.dot(h, w_ref[...], preferred_element_type=jnp.float32)
    y2_ref[...] = jnp.where(rows < N, y2, 0.0)

  return pl.pallas_call(
      body,
      grid=(NBLK,),
      in_specs=[
          pl.BlockSpec((NCORES, RB, DH), lambda i: (0, i, 0)),
          pl.BlockSpec((RB, DH), lambda i: (i, 0)),
          pl.BlockSpec((RB, 1), lambda i: (i, 0)),
          pl.BlockSpec((1, DH), lambda i: (0, 0)),
          pl.BlockSpec((DH, DH), lambda i: (0, 0)),
      ],
      out_specs=[
          pl.BlockSpec((RB, DH), lambda i: (i, 0)),
          pl.BlockSpec((RB, DH), lambda i: (i, 0)),
      ],
      out_shape=[
          jax.ShapeDtypeStruct((NPAD, DH), jnp.float32),
          jax.ShapeDtypeStruct((NPAD, DH), jnp.float32),
      ],
  )(accp, y, degp_d, bias, Wn)


def _tc_last(accp, y, degp_d, bias):
  def body(a_ref, y_ref, dg_ref, b_ref, h_ref):
    i = pl.program_id(0)
    rows = i * RB + lax.broadcasted_iota(jnp.int32, (RB, 1), 0)
    dinv = _dinv_block(dg_ref)
    s = a_ref[0] + a_ref[1] + y_ref[...]
    h = jnp.maximum(dinv * s + b_ref[...], 0.0)
    h_ref[...] = jnp.where(rows < N, h, 0.0)

  return pl.pallas_call(
      body,
      grid=(NBLK,),
      in_specs=[
          pl.BlockSpec((NCORES, RB, DH), lambda i: (0, i, 0)),
          pl.BlockSpec((RB, DH), lambda i: (i, 0)),
          pl.BlockSpec((RB, 1), lambda i: (i, 0)),
          pl.BlockSpec((1, DH), lambda i: (0, 0)),
      ],
      out_specs=pl.BlockSpec((RB, DH), lambda i: (i, 0)),
      out_shape=jax.ShapeDtypeStruct((NPAD, DH), jnp.float32),
  )(accp, y, degp_d, bias)


def _tc_pool(hs, batchp):
  njk = len(hs)

  def body(b_ref, *rest):
    h_refs = rest[:njk]
    out_ref = rest[njk]
    i = pl.program_id(0)

    @pl.when(i == 0)
    def _init():
      out_ref[...] = jnp.full((B, njk * DH), -jnp.inf, jnp.float32)

    rows = i * RB + lax.broadcasted_iota(jnp.int32, (RB, 1), 0)
    valid = rows < N
    bb = b_ref[...]
    hcat = jnp.concatenate([h[...] for h in h_refs], axis=1)
    b_lo = jnp.min(jnp.where(valid, bb, B - 1))
    b_hi = jnp.max(jnp.where(valid, bb, 0))

    def bbody(b, _):
      m = (bb == b) & valid
      mx = jnp.max(jnp.where(m, hcat, -jnp.inf), axis=0, keepdims=True)
      out_ref[pl.ds(b, 1), :] = jnp.maximum(out_ref[pl.ds(b, 1), :], mx)
      return 0
    lax.fori_loop(b_lo, b_hi + 1, bbody, 0)

  return pl.pallas_call(
      body,
      grid=(NBLK,),
      in_specs=[pl.BlockSpec((RB, 1), lambda i: (i, 0))] +
               [pl.BlockSpec((RB, DH), lambda i: (i, 0))] * njk,
      out_specs=pl.BlockSpec((B, njk * DH), lambda i: (0, 0)),
      out_shape=jax.ShapeDtypeStruct((B, njk * DH), jnp.float32),
  )(batchp, *hs)


def _tc_head(pooled, pi, Wg, bg, gg, gb, Wp, bp, pg, pb, Wf, bf, fg, fb):
  def ln(v, g, beta):
    mu = jnp.mean(v, axis=-1, keepdims=True)
    var = jnp.mean((v - mu) ** 2, axis=-1, keepdims=True)
    return (v - mu) / jnp.sqrt(var + 1e-5) * g + beta

  def body(pool_ref, pi_ref, wg_ref, bg_ref, gg_ref, gb_ref,
           wp_ref, bp_ref, pg_ref, pb_ref, wf_ref, bf_ref, fg_ref, fb_ref,
           out_ref):
    g = jnp.dot(pool_ref[...], wg_ref[...],
                preferred_element_type=jnp.float32) + bg_ref[...]
    g = jnp.maximum(ln(g, gg_ref[...], gb_ref[...]), 0.0)
    p = jnp.dot(pi_ref[...], wp_ref[...],
                preferred_element_type=jnp.float32) + bp_ref[...]
    p = jnp.maximum(ln(p, pg_ref[...], pb_ref[...]), 0.0)
    h = jnp.concatenate([g, p], axis=1)
    o = jnp.dot(h, wf_ref[...], preferred_element_type=jnp.float32) \
        + bf_ref[...]
    out_ref[...] = ln(o, fg_ref[...], fb_ref[...])

  args = (pooled, pi, Wg, bg, gg, gb, Wp, bp, pg, pb, Wf, bf, fg, fb)

  def _spec(a):
    zeros = tuple(0 for _ in a.shape)
    return pl.BlockSpec(a.shape, lambda z=zeros: z)

  return pl.pallas_call(
      body,
      in_specs=[_spec(a) for a in args],
      out_specs=pl.BlockSpec((B, Wf.shape[1]), lambda: (0, 0)),
      out_shape=jax.ShapeDtypeStruct((B, Wf.shape[1]), jnp.float32),
  )(*args)


def kernel(x, edge_index, edge_type, batch, pi, W0, b0, Wh, bh, Wg, bg,
           g_gamma, g_beta, Wp, bp, p_gamma, p_beta, Wf, bf, f_gamma,
           f_beta):
  src = edge_index[0].astype(jnp.int32)
  dst = edge_index[1].astype(jnp.int32)
  et0 = edge_type[:, 0].astype(jnp.int32)
  et1 = edge_type[:, 1].astype(jnp.int32)
  xp = jnp.pad(x, ((0, NPAD - N), (0, 0)))
  batchp = jnp.pad(batch.astype(jnp.int32), (0, NPAD - N)).reshape(NPAD, 1)

  srcc, dstc, counts, degp = _sc_prep(src, dst, et0, et1)

  hs = []
  for d in range(NDIM):
    dg = jnp.sum(degp[d], axis=0).reshape(NPAD, 1)
    y = _tc_first(xp, W0[d], dg)
    for l in range(NL):
      accp = _sc_scatter(d, y, srcc, dstc, counts)
      bias = (b0[d] if l == 0 else bh[d, l - 1]).reshape(1, DH)
      if l < NL - 1:
        h, y = _tc_mid(accp, y, dg, bias, Wh[d, l])
      else:
        h = _tc_last(accp, y, dg, bias)
      hs.append(h)

  pooled = _tc_pool(hs, batchp)
  out = _tc_head(pooled, pi,
                 Wg, bg.reshape(1, -1), g_gamma.reshape(1, -1),
                 g_beta.reshape(1, -1),
                 Wp, bp.reshape(1, -1), p_gamma.reshape(1, -1),
                 p_beta.reshape(1, -1),
                 Wf, bf.reshape(1, -1), f_gamma.reshape(1, -1),
                 f_beta.reshape(1, -1))
  return out


# speedup vs baseline: 25.5104x; 1.0064x over previous
import functools

import jax
import jax.numpy as jnp
from jax import lax
from jax.experimental import pallas as pl
from jax.experimental.pallas import tpu as pltpu
from jax.experimental.pallas import tpu_sc as plsc

N = 10000
E = 320000
NDIM = 2
NL = 3
DH = 128
B = 64

NCORES = 2
NSUB = 16
NWORK = NCORES * NSUB
EPW = E // NWORK
G = 64
CHUNKS_CAP = 160
TRASH = N

NROWS_SC = 10112
RPT = NROWS_SC // NSUB

RB = 256
NPAD = 10240
NBLK = NPAD // RB

_MESH = dict(core_axis_name="c", subcore_axis_name="s",
             num_cores=NCORES, num_subcores=NSUB)


def _sc_prep(src, dst, et0, et1):
  mesh = plsc.VectorSubcoreMesh(**_MESH)
  out_type = (
      jax.ShapeDtypeStruct((NDIM, NWORK, CHUNKS_CAP, G), jnp.int32),
      jax.ShapeDtypeStruct((NDIM, NWORK, CHUNKS_CAP, G), jnp.int32),
      jax.ShapeDtypeStruct((NDIM, NWORK, 16), jnp.int32),
      jax.ShapeDtypeStruct((NDIM, NWORK, NPAD), jnp.float32),
  )
  scratch = [
      pltpu.VMEM((EPW,), jnp.int32),
      pltpu.VMEM((EPW,), jnp.int32),
      pltpu.VMEM((EPW,), jnp.int32),
      pltpu.VMEM((CHUNKS_CAP * G,), jnp.int32),
      pltpu.VMEM((CHUNKS_CAP * G,), jnp.int32),
      pltpu.VMEM((CHUNKS_CAP, G), jnp.int32),
      pltpu.VMEM((CHUNKS_CAP, G), jnp.int32),
      pltpu.VMEM((16,), jnp.int32),
      pltpu.VMEM((NPAD,), jnp.float32),
  ]

  @functools.partial(
      pl.kernel, out_type=out_type, mesh=mesh, scratch_types=scratch,
      compiler_params=pltpu.CompilerParams(needs_layout_passes=False),
  )
  def prep(src_h, dst_h, et0_h, et1_h, srcc_h, dstc_h, cnt_h, degp_h,
           se, de, ee, sflat, dflat, s2d, d2d, cnt_v, degl):
    cid = lax.axis_index("c")
    sid = lax.axis_index("s")
    wid = cid * NSUB + sid
    base = wid * EPW
    pltpu.sync_copy(src_h.at[pl.ds(base, EPW)], se)
    pltpu.sync_copy(dst_h.at[pl.ds(base, EPW)], de)

    ii = lax.iota(jnp.int32, 16)
    dummy = jnp.full((16,), TRASH, jnp.int32)
    fones = jnp.ones((16,), jnp.float32)
    z16 = jnp.zeros((16,), jnp.float32)

    for d, et_h in enumerate((et0_h, et1_h)):
      pltpu.sync_copy(et_h.at[pl.ds(base, EPW)], ee)

      @pl.loop(0, NPAD // 16)
      def _zdeg(i):
        degl[pl.ds(i * 16, 16)] = z16

      def cbody(i, c):
        sl = pl.ds(i * 16, 16)
        s16 = se[sl]
        d16 = de[sl]
        e16 = ee[sl]
        m = e16 == 1
        mi = m.astype(jnp.int32)
        pos = c + plsc.cumsum(mi) - 1
        plsc.store_scatter(sflat, [pos], s16, mask=m)
        plsc.store_scatter(dflat, [pos], d16, mask=m)
        plsc.addupdate_scatter(degl, [d16], fones, mask=m)
        return c + jnp.sum(mi)
      c = lax.fori_loop(0, EPW // 16, cbody, jnp.int32(0), unroll=2)

      for t in range(G // 16):
        idx = c + t * 16 + ii
        plsc.store_scatter(sflat, [idx], dummy)
        plsc.store_scatter(dflat, [idx], dummy)
      nch = (c + G - 1) // G

      def copybody(j, _):
        r = j // (G // 16)
        cc = (j % (G // 16)) * 16
        s2d[r, pl.ds(cc, 16)] = sflat[pl.ds(j * 16, 16)]
        d2d[r, pl.ds(cc, 16)] = dflat[pl.ds(j * 16, 16)]
        return 0
      lax.fori_loop(0, nch * (G // 16), copybody, 0)

      cnt_v[...] = jnp.full((16,), nch, jnp.int32)
      pltpu.sync_copy(cnt_v, cnt_h.at[d, wid])
      pltpu.sync_copy(s2d, srcc_h.at[d, wid])
      pltpu.sync_copy(d2d, dstc_h.at[d, wid])
      pltpu.sync_copy(degl, degp_h.at[d, wid])

  return prep(src, dst, et0, et1)


_NROW = 5
_NIDX = 10


def _sc_scatter(d, y, srcc, dstc, counts, width=DH):
  mesh = plsc.VectorSubcoreMesh(**_MESH)
  out_type = jax.ShapeDtypeStruct((NCORES, NPAD, width), jnp.float32)
  scratch = [
      pltpu.VMEM((_NIDX, G), jnp.int32),
      pltpu.VMEM((_NIDX, G), jnp.int32),
      pltpu.VMEM((16,), jnp.int32),
      pltpu.VMEM((32, width), jnp.float32),
      pltpu.VMEM_SHARED((NROWS_SC, width), jnp.float32),
  ] + [pltpu.VMEM((G, width), jnp.float32) for _ in range(_NROW)] \
    + [pltpu.SemaphoreType.DMA for _ in range(2 * _NROW + _NIDX)]

  @functools.partial(
      pl.kernel, out_type=out_type, mesh=mesh, scratch_types=scratch,
      compiler_params=pltpu.CompilerParams(needs_layout_passes=False),
  )
  def scat(y_h, srcc_h, dstc_h, cnt_h, acc_h, sidx, didx, cv, zrow, accsh,
           *bufsem):
    rows = bufsem[:_NROW]
    gsem = bufsem[_NROW:2 * _NROW]
    ssem = bufsem[2 * _NROW:3 * _NROW]
    isem = bufsem[3 * _NROW:]
    cid = lax.axis_index("c")
    sid = lax.axis_index("s")
    wid = cid * NSUB + sid
    pltpu.sync_copy(cnt_h.at[d, wid], cv)

    def start_idx(j, b4):
      pltpu.async_copy(srcc_h.at[d, wid, j], sidx.at[b4], isem[b4])
      pltpu.async_copy(dstc_h.at[d, wid, j], didx.at[b4], isem[b4])

    def wait_idx(j, b4):
      pltpu.make_async_copy(srcc_h.at[d, wid, j], sidx.at[b4],
                            isem[b4]).wait()
      pltpu.make_async_copy(dstc_h.at[d, wid, j], didx.at[b4],
                            isem[b4]).wait()

    def start_gather(j, b, b4):
      pltpu.async_copy(y_h.at[sidx.at[b4]], rows[b], gsem[b])

    def wait_gather(b, b4):
      pltpu.make_async_copy(y_h.at[sidx.at[b4]], rows[b], gsem[b]).wait()

    def start_scatter(b, b4):
      pltpu.async_copy(rows[b], accsh.at[didx.at[b4]], ssem[b], add=True)

    def wait_scatter(b, b4):
      pltpu.make_async_copy(rows[b], accsh.at[didx.at[b4]], ssem[b]).wait()

    z16 = jnp.zeros((16,), jnp.float32)

    @pl.loop(0, 32)
    def _zfill(i):
      for c8 in range(width // 16):
        zrow[i, pl.ds(c8 * 16, 16)] = z16

    r0 = sid * RPT
    for k in range(RPT // 32):
      pltpu.sync_copy(zrow, accsh.at[pl.ds(r0 + k * 32, 32)])
    rem = RPT % 32
    if rem:
      pltpu.sync_copy(zrow.at[pl.ds(0, rem)],
                      accsh.at[pl.ds(r0 + (RPT // 32) * 32, rem)])
    plsc.subcore_barrier()

    nch = cv[...][0]

    for b in range(_NIDX):
      @pl.when(b < nch)
      def _pi(b=b):
        start_idx(b, b)
    for b in range(_NROW):
      @pl.when(b < nch)
      def _pg(b=b):
        wait_idx(b, b)
        start_gather(b, b, b)

    def outer(k, _):
      for u in range(_NIDX):
        j = k * _NIDX + u
        b = u % _NROW
        b4 = u

        @pl.when(j < nch)
        def _work(b=b, j=j, b4=b4):
          wait_gather(b, b4)
          start_scatter(b, b4)

        @pl.when(j + _NROW < nch)
        def _next(b=b, j=j, b4=b4):
          wait_scatter(b, b4)
          @pl.when(j + _NIDX < nch)
          def _refill(j=j, b4=b4):
            start_idx(j + _NIDX, b4)
          bn = (b4 + _NROW) % _NIDX
          wait_idx(j + _NROW, bn)
          start_gather(j + _NROW, b, bn)
      return 0
    lax.fori_loop(0, (nch + _NIDX - 1) // _NIDX, outer, 0)
    for b in range(_NROW):
      @pl.when(b < nch)
      def _drain(b=b):
        pltpu.make_async_copy(rows[b], accsh.at[didx.at[0]], ssem[b]).wait()

    plsc.subcore_barrier()
    pltpu.sync_copy(accsh.at[pl.ds(r0, RPT)],
                    acc_h.at[cid, pl.ds(r0, RPT)])

  return scat(y, srcc, dstc, counts)


def _dinv_block(dg_ref):
  deg = 1.0 + dg_ref[...]
  return 1.0 / jnp.sqrt(deg)


def _tc_first(xp, W, degp_d):
  def body(x_ref, w_ref, dg_ref, y_ref):
    i = pl.program_id(0)
    rows = i * RB + lax.broadcasted_iota(jnp.int32, (RB, 1), 0)
    dinv = _dinv_block(dg_ref)
    xw = jnp.dot(x_ref[...], w_ref[...], preferred_element_type=jnp.float32)
    y_ref[...] = jnp.where(rows < N, dinv * xw, 0.0)

  return pl.pallas_call(
      body,
      grid=(NBLK,),
      in_specs=[
          pl.BlockSpec((RB, DH), lambda i: (i, 0)),
          pl.BlockSpec((DH, DH), lambda i: (0, 0)),
          pl.BlockSpec((RB, 1), lambda i: (i, 0)),
      ],
      out_specs=pl.BlockSpec((RB, DH), lambda i: (i, 0)),
      out_shape=jax.ShapeDtypeStruct((NPAD, DH), jnp.float32),
  )(xp, W, degp_d)


def _tc_mid(accp, y, degp_d, bias, Wn):
  def body(a_ref, y_ref, dg_ref, b_ref, w_ref, h_ref, y2_ref):
    i = pl.program_id(0)
    rows = i * RB + lax.broadcasted_iota(jnp.int32, (RB, 1), 0)
    dinv = _dinv_block(dg_ref)
    s = a_ref[0] + a_ref[1] + y_ref[...]
    h = jnp.maximum(dinv * s + b_ref[...], 0.0)
    h = jnp.where(rows < N, h, 0.0)
    h_ref[...] = h
    y2 = dinv * jnp.dot(h, w_ref[...], preferred_element_type=jnp.float32)
    y2_ref[...] = jnp.where(rows < N, y2, 0.0)

  return pl.pallas_call(
      body,
      grid=(NBLK,),
      in_specs=[
          pl.BlockSpec((NCORES, RB, DH), lambda i: (0, i, 0)),
          pl.BlockSpec((RB, DH), lambda i: (i, 0)),
          pl.BlockSpec((RB, 1), lambda i: (i, 0)),
          pl.BlockSpec((1, DH), lambda i: (0, 0)),
          pl.BlockSpec((DH, DH), lambda i: (0, 0)),
      ],
      out_specs=[
          pl.BlockSpec((RB, DH), lambda i: (i, 0)),
          pl.BlockSpec((RB, DH), lambda i: (i, 0)),
      ],
      out_shape=[
          jax.ShapeDtypeStruct((NPAD, DH), jnp.float32),
          jax.ShapeDtypeStruct((NPAD, DH), jnp.float32),
      ],
  )(accp, y, degp_d, bias, Wn)


def _tc_last(accp, y, degp_d, bias):
  def body(a_ref, y_ref, dg_ref, b_ref, h_ref):
    i = pl.program_id(0)
    rows = i * RB + lax.broadcasted_iota(jnp.int32, (RB, 1), 0)
    dinv = _dinv_block(dg_ref)
    s = a_ref[0] + a_ref[1] + y_ref[...]
    h = jnp.maximum(dinv * s + b_ref[...], 0.0)
    h_ref[...] = jnp.where(rows < N, h, 0.0)

  return pl.pallas_call(
      body,
      grid=(NBLK,),
      in_specs=[
          pl.BlockSpec((NCORES, RB, DH), lambda i: (0, i, 0)),
          pl.BlockSpec((RB, DH), lambda i: (i, 0)),
          pl.BlockSpec((RB, 1), lambda i: (i, 0)),
          pl.BlockSpec((1, DH), lambda i: (0, 0)),
      ],
      out_specs=pl.BlockSpec((RB, DH), lambda i: (i, 0)),
      out_shape=jax.ShapeDtypeStruct((NPAD, DH), jnp.float32),
  )(accp, y, degp_d, bias)


def _tc_pool(hs, batchp):
  njk = len(hs)

  def body(b_ref, *rest):
    h_refs = rest[:njk]
    out_ref = rest[njk]
    i = pl.program_id(0)

    @pl.when(i == 0)
    def _init():
      out_ref[...] = jnp.full((B, njk * DH), -jnp.inf, jnp.float32)

    rows = i * RB + lax.broadcasted_iota(jnp.int32, (RB, 1), 0)
    valid = rows < N
    bb = b_ref[...]
    hcat = jnp.concatenate([h[...] for h in h_refs], axis=1)
    b_lo = jnp.min(jnp.where(valid, bb, B - 1))
    b_hi = jnp.max(jnp.where(valid, bb, 0))

    def bbody(b, _):
      m = (bb == b) & valid
      mx = jnp.max(jnp.where(m, hcat, -jnp.inf), axis=0, keepdims=True)
      out_ref[pl.ds(b, 1), :] = jnp.maximum(out_ref[pl.ds(b, 1), :], mx)
      return 0
    lax.fori_loop(b_lo, b_hi + 1, bbody, 0)

  return pl.pallas_call(
      body,
      grid=(NBLK,),
      in_specs=[pl.BlockSpec((RB, 1), lambda i: (i, 0))] +
               [pl.BlockSpec((RB, DH), lambda i: (i, 0))] * njk,
      out_specs=pl.BlockSpec((B, njk * DH), lambda i: (0, 0)),
      out_shape=jax.ShapeDtypeStruct((B, njk * DH), jnp.float32),
  )(batchp, *hs)


def _tc_head(pooled, pi, Wg, bg, gg, gb, Wp, bp, pg, pb, Wf, bf, fg, fb):
  def ln(v, g, beta):
    mu = jnp.mean(v, axis=-1, keepdims=True)
    var = jnp.mean((v - mu) ** 2, axis=-1, keepdims=True)
    return (v - mu) / jnp.sqrt(var + 1e-5) * g + beta

  def body(pool_ref, pi_ref, wg_ref, bg_ref, gg_ref, gb_ref,
           wp_ref, bp_ref, pg_ref, pb_ref, wf_ref, bf_ref, fg_ref, fb_ref,
           out_ref):
    g = jnp.dot(pool_ref[...], wg_ref[...],
                preferred_element_type=jnp.float32) + bg_ref[...]
    g = jnp.maximum(ln(g, gg_ref[...], gb_ref[...]), 0.0)
    p = jnp.dot(pi_ref[...], wp_ref[...],
                preferred_element_type=jnp.float32) + bp_ref[...]
    p = jnp.maximum(ln(p, pg_ref[...], pb_ref[...]), 0.0)
    h = jnp.concatenate([g, p], axis=1)
    o = jnp.dot(h, wf_ref[...], preferred_element_type=jnp.float32) \
        + bf_ref[...]
    out_ref[...] = ln(o, fg_ref[...], fb_ref[...])

  args = (pooled, pi, Wg, bg, gg, gb, Wp, bp, pg, pb, Wf, bf, fg, fb)

  def _spec(a):
    zeros = tuple(0 for _ in a.shape)
    return pl.BlockSpec(a.shape, lambda z=zeros: z)

  return pl.pallas_call(
      body,
      in_specs=[_spec(a) for a in args],
      out_specs=pl.BlockSpec((B, Wf.shape[1]), lambda: (0, 0)),
      out_shape=jax.ShapeDtypeStruct((B, Wf.shape[1]), jnp.float32),
  )(*args)


def kernel(x, edge_index, edge_type, batch, pi, W0, b0, Wh, bh, Wg, bg,
           g_gamma, g_beta, Wp, bp, p_gamma, p_beta, Wf, bf, f_gamma,
           f_beta):
  src = edge_index[0].astype(jnp.int32)
  dst = edge_index[1].astype(jnp.int32)
  et0 = edge_type[:, 0].astype(jnp.int32)
  et1 = edge_type[:, 1].astype(jnp.int32)
  xp = jnp.pad(x, ((0, NPAD - N), (0, 0)))
  batchp = jnp.pad(batch.astype(jnp.int32), (0, NPAD - N)).reshape(NPAD, 1)

  srcc, dstc, counts, degp = _sc_prep(src, dst, et0, et1)

  hs = []
  for d in range(NDIM):
    dg = jnp.sum(degp[d], axis=0).reshape(NPAD, 1)
    y = _tc_first(xp, W0[d], dg)
    for l in range(NL):
      accp = _sc_scatter(d, y, srcc, dstc, counts)
      bias = (b0[d] if l == 0 else bh[d, l - 1]).reshape(1, DH)
      if l < NL - 1:
        h, y = _tc_mid(accp, y, dg, bias, Wh[d, l])
      else:
        h = _tc_last(accp, y, dg, bias)
      hs.append(h)

  pooled = _tc_pool(hs, batchp)
  out = _tc_head(pooled, pi,
                 Wg, bg.reshape(1, -1), g_gamma.reshape(1, -1),
                 g_beta.reshape(1, -1),
                 Wp, bp.reshape(1, -1), p_gamma.reshape(1, -1),
                 p_beta.reshape(1, -1),
                 Wf, bf.reshape(1, -1), f_gamma.reshape(1, -1),
                 f_beta.reshape(1, -1))
  return out


# speedup vs baseline: 26.5335x; 1.0401x over previous
import functools

import jax
import jax.numpy as jnp
from jax import lax
from jax.experimental import pallas as pl
from jax.experimental.pallas import tpu as pltpu
from jax.experimental.pallas import tpu_sc as plsc

N = 10000
E = 320000
NDIM = 2
NL = 3
DH = 128
B = 64

NCORES = 2
NSUB = 16
NWORK = NCORES * NSUB
EPW = E // NWORK
G = 64
CHUNKS_CAP = 160
TRASH = N

NROWS_SC = 10112
RPT = NROWS_SC // NSUB

RB = 256
NPAD = 10240
NBLK = NPAD // RB

_MESH = dict(core_axis_name="c", subcore_axis_name="s",
             num_cores=NCORES, num_subcores=NSUB)


def _sc_prep(src, dst, et0, et1):
  mesh = plsc.VectorSubcoreMesh(**_MESH)
  out_type = (
      jax.ShapeDtypeStruct((NDIM, NWORK, CHUNKS_CAP, G), jnp.int32),
      jax.ShapeDtypeStruct((NDIM, NWORK, CHUNKS_CAP, G), jnp.int32),
      jax.ShapeDtypeStruct((NDIM, NWORK, 16), jnp.int32),
      jax.ShapeDtypeStruct((NDIM, NWORK, NPAD), jnp.float32),
  )
  scratch = [
      pltpu.VMEM((EPW,), jnp.int32),
      pltpu.VMEM((EPW,), jnp.int32),
      pltpu.VMEM((EPW,), jnp.int32),
      pltpu.VMEM((CHUNKS_CAP * G,), jnp.int32),
      pltpu.VMEM((CHUNKS_CAP * G,), jnp.int32),
      pltpu.VMEM((CHUNKS_CAP, G), jnp.int32),
      pltpu.VMEM((CHUNKS_CAP, G), jnp.int32),
      pltpu.VMEM((16,), jnp.int32),
      pltpu.VMEM((NPAD,), jnp.float32),
  ]

  @functools.partial(
      pl.kernel, out_type=out_type, mesh=mesh, scratch_types=scratch,
      compiler_params=pltpu.CompilerParams(needs_layout_passes=False),
  )
  def prep(src_h, dst_h, et0_h, et1_h, srcc_h, dstc_h, cnt_h, degp_h,
           se, de, ee, sflat, dflat, s2d, d2d, cnt_v, degl):
    cid = lax.axis_index("c")
    sid = lax.axis_index("s")
    wid = cid * NSUB + sid
    base = wid * EPW
    pltpu.sync_copy(src_h.at[pl.ds(base, EPW)], se)
    pltpu.sync_copy(dst_h.at[pl.ds(base, EPW)], de)

    ii = lax.iota(jnp.int32, 16)
    dummy = jnp.full((16,), TRASH, jnp.int32)
    fones = jnp.ones((16,), jnp.float32)
    z16 = jnp.zeros((16,), jnp.float32)

    for d, et_h in enumerate((et0_h, et1_h)):
      pltpu.sync_copy(et_h.at[pl.ds(base, EPW)], ee)

      @pl.loop(0, NPAD // 16)
      def _zdeg(i):
        degl[pl.ds(i * 16, 16)] = z16

      def cbody(i, c):
        sl = pl.ds(i * 16, 16)
        s16 = se[sl]
        d16 = de[sl]
        e16 = ee[sl]
        m = e16 == 1
        mi = m.astype(jnp.int32)
        pos = c + plsc.cumsum(mi) - 1
        plsc.store_scatter(sflat, [pos], s16, mask=m)
        plsc.store_scatter(dflat, [pos], d16, mask=m)
        plsc.addupdate_scatter(degl, [d16], fones, mask=m)
        return c + jnp.sum(mi)
      c = lax.fori_loop(0, EPW // 16, cbody, jnp.int32(0), unroll=2)

      for t in range(G // 16):
        idx = c + t * 16 + ii
        plsc.store_scatter(sflat, [idx], dummy)
        plsc.store_scatter(dflat, [idx], dummy)
      nch = (c + G - 1) // G

      def copybody(j, _):
        r = j // (G // 16)
        cc = (j % (G // 16)) * 16
        s2d[r, pl.ds(cc, 16)] = sflat[pl.ds(j * 16, 16)]
        d2d[r, pl.ds(cc, 16)] = dflat[pl.ds(j * 16, 16)]
        return 0
      lax.fori_loop(0, nch * (G // 16), copybody, 0)

      cnt_v[...] = jnp.full((16,), nch, jnp.int32)
      pltpu.sync_copy(cnt_v, cnt_h.at[d, wid])
      pltpu.sync_copy(s2d, srcc_h.at[d, wid])
      pltpu.sync_copy(d2d, dstc_h.at[d, wid])
      pltpu.sync_copy(degl, degp_h.at[d, wid])

  return prep(src, dst, et0, et1)


_NROW = 5
_NIDX = 10


def _sc_scatter(d, y, srcc, dstc, counts, width=DH):
  mesh = plsc.VectorSubcoreMesh(**_MESH)
  out_type = jax.ShapeDtypeStruct((NCORES, NPAD, width), jnp.float32)
  scratch = [
      pltpu.VMEM((_NIDX, G), jnp.int32),
      pltpu.VMEM((_NIDX, G), jnp.int32),
      pltpu.VMEM((16,), jnp.int32),
      pltpu.VMEM((32, width), jnp.float32),
      pltpu.VMEM_SHARED((NROWS_SC, width), jnp.float32),
  ] + [pltpu.VMEM((G, width), jnp.float32) for _ in range(_NROW)] \
    + [pltpu.SemaphoreType.DMA for _ in range(2 * _NROW + _NIDX)]

  @functools.partial(
      pl.kernel, out_type=out_type, mesh=mesh, scratch_types=scratch,
      compiler_params=pltpu.CompilerParams(needs_layout_passes=False),
  )
  def scat(y_h, srcc_h, dstc_h, cnt_h, acc_h, sidx, didx, cv, zrow, accsh,
           *bufsem):
    rows = bufsem[:_NROW]
    gsem = bufsem[_NROW:2 * _NROW]
    ssem = bufsem[2 * _NROW:3 * _NROW]
    isem = bufsem[3 * _NROW:]
    cid = lax.axis_index("c")
    sid = lax.axis_index("s")
    wid = cid * NSUB + sid
    pltpu.sync_copy(cnt_h.at[d, wid], cv)

    def start_idx(j, b4):
      pltpu.async_copy(srcc_h.at[d, wid, j], sidx.at[b4], isem[b4])
      pltpu.async_copy(dstc_h.at[d, wid, j], didx.at[b4], isem[b4])

    def wait_idx(j, b4):
      pltpu.make_async_copy(srcc_h.at[d, wid, j], sidx.at[b4],
                            isem[b4]).wait()
      pltpu.make_async_copy(dstc_h.at[d, wid, j], didx.at[b4],
                            isem[b4]).wait()

    def start_gather(j, b, b4):
      pltpu.async_copy(y_h.at[sidx.at[b4]], rows[b], gsem[b])

    def wait_gather(b, b4):
      pltpu.make_async_copy(y_h.at[sidx.at[b4]], rows[b], gsem[b]).wait()

    def start_scatter(b, b4):
      pltpu.async_copy(rows[b], accsh.at[didx.at[b4]], ssem[b], add=True)

    def wait_scatter(b, b4):
      pltpu.make_async_copy(rows[b], accsh.at[didx.at[b4]], ssem[b]).wait()

    z16 = jnp.zeros((16,), jnp.float32)

    @pl.loop(0, 32)
    def _zfill(i):
      for c8 in range(width // 16):
        zrow[i, pl.ds(c8 * 16, 16)] = z16

    r0 = sid * RPT
    for k in range(RPT // 32):
      pltpu.sync_copy(zrow, accsh.at[pl.ds(r0 + k * 32, 32)])
    rem = RPT % 32
    if rem:
      pltpu.sync_copy(zrow.at[pl.ds(0, rem)],
                      accsh.at[pl.ds(r0 + (RPT // 32) * 32, rem)])
    plsc.subcore_barrier()

    nch = cv[...][0]

    for b in range(_NIDX):
      @pl.when(b < nch)
      def _pi(b=b):
        start_idx(b, b)
    for b in range(_NROW):
      @pl.when(b < nch)
      def _pg(b=b):
        wait_idx(b, b)
        start_gather(b, b, b)

    def outer(k, _):
      for u in range(_NIDX):
        j = k * _NIDX + u
        b = u % _NROW
        b4 = u

        @pl.when(j < nch)
        def _work(b=b, j=j, b4=b4):
          wait_gather(b, b4)
          start_scatter(b, b4)

        @pl.when(j + _NROW < nch)
        def _next(b=b, j=j, b4=b4):
          wait_scatter(b, b4)
          @pl.when(j + _NIDX < nch)
          def _refill(j=j, b4=b4):
            start_idx(j + _NIDX, b4)
          bn = (b4 + _NROW) % _NIDX
          wait_idx(j + _NROW, bn)
          start_gather(j + _NROW, b, bn)
      return 0
    lax.fori_loop(0, (nch + _NIDX - 1) // _NIDX, outer, 0)
    for b in range(_NROW):
      @pl.when(b < nch)
      def _drain(b=b):
        pltpu.make_async_copy(rows[b], accsh.at[didx.at[0]], ssem[b]).wait()

    plsc.subcore_barrier()
    pltpu.sync_copy(accsh.at[pl.ds(r0, RPT)],
                    acc_h.at[cid, pl.ds(r0, RPT)])

  return scat(y, srcc, dstc, counts)


def _dinv_block(dg_ref):
  deg = 1.0 + dg_ref[...]
  return 1.0 / jnp.sqrt(deg)


def _tc_first(xp, W, degp_d):
  def body(x_ref, w_ref, dg_ref, y_ref):
    i = pl.program_id(0)
    rows = i * RB + lax.broadcasted_iota(jnp.int32, (RB, 1), 0)
    dinv = _dinv_block(dg_ref)
    xw = jnp.dot(x_ref[...], w_ref[...], preferred_element_type=jnp.float32)
    y_ref[...] = jnp.where(rows < N, dinv * xw, 0.0)

  return pl.pallas_call(
      body,
      grid=(NBLK,),
      in_specs=[
          pl.BlockSpec((RB, DH), lambda i: (i, 0)),
          pl.BlockSpec((DH, DH), lambda i: (0, 0)),
          pl.BlockSpec((RB, 1), lambda i: (i, 0)),
      ],
      out_specs=pl.BlockSpec((RB, DH), lambda i: (i, 0)),
      out_shape=jax.ShapeDtypeStruct((NPAD, DH), jnp.float32),
  )(xp, W, degp_d)


def _pool_block(i, h, bb, rows, pp_ref):
  @pl.when(i == 0)
  def _init():
    pp_ref[...] = jnp.full((B, DH), -jnp.inf, jnp.float32)

  valid = rows < N
  b_lo = jnp.min(jnp.where(valid, bb, B - 1))
  b_hi = jnp.max(jnp.where(valid, bb, 0))

  def bbody(b, _):
    m = (bb == b) & valid
    mx = jnp.max(jnp.where(m, h, -jnp.inf), axis=0, keepdims=True)
    pp_ref[pl.ds(b, 1), :] = jnp.maximum(pp_ref[pl.ds(b, 1), :], mx)
    return 0
  lax.fori_loop(b_lo, b_hi + 1, bbody, 0)


def _tc_mid(accp, y, degp_d, bias, Wn, batchp):
  def body(a_ref, y_ref, dg_ref, b_ref, w_ref, bt_ref, y2_ref, pp_ref):
    i = pl.program_id(0)
    rows = i * RB + lax.broadcasted_iota(jnp.int32, (RB, 1), 0)
    dinv = _dinv_block(dg_ref)
    s = a_ref[0] + a_ref[1] + y_ref[...]
    h = jnp.maximum(dinv * s + b_ref[...], 0.0)
    h = jnp.where(rows < N, h, 0.0)
    y2 = dinv * jnp.dot(h, w_ref[...], preferred_element_type=jnp.float32)
    y2_ref[...] = jnp.where(rows < N, y2, 0.0)
    _pool_block(i, h, bt_ref[...], rows, pp_ref)

  return pl.pallas_call(
      body,
      grid=(NBLK,),
      in_specs=[
          pl.BlockSpec((NCORES, RB, DH), lambda i: (0, i, 0)),
          pl.BlockSpec((RB, DH), lambda i: (i, 0)),
          pl.BlockSpec((RB, 1), lambda i: (i, 0)),
          pl.BlockSpec((1, DH), lambda i: (0, 0)),
          pl.BlockSpec((DH, DH), lambda i: (0, 0)),
          pl.BlockSpec((RB, 1), lambda i: (i, 0)),
      ],
      out_specs=[
          pl.BlockSpec((RB, DH), lambda i: (i, 0)),
          pl.BlockSpec((B, DH), lambda i: (0, 0)),
      ],
      out_shape=[
          jax.ShapeDtypeStruct((NPAD, DH), jnp.float32),
          jax.ShapeDtypeStruct((B, DH), jnp.float32),
      ],
  )(accp, y, degp_d, bias, Wn, batchp)


def _tc_last(accp, y, degp_d, bias, batchp):
  def body(a_ref, y_ref, dg_ref, b_ref, bt_ref, pp_ref):
    i = pl.program_id(0)
    rows = i * RB + lax.broadcasted_iota(jnp.int32, (RB, 1), 0)
    dinv = _dinv_block(dg_ref)
    s = a_ref[0] + a_ref[1] + y_ref[...]
    h = jnp.maximum(dinv * s + b_ref[...], 0.0)
    h = jnp.where(rows < N, h, 0.0)
    _pool_block(i, h, bt_ref[...], rows, pp_ref)

  return pl.pallas_call(
      body,
      grid=(NBLK,),
      in_specs=[
          pl.BlockSpec((NCORES, RB, DH), lambda i: (0, i, 0)),
          pl.BlockSpec((RB, DH), lambda i: (i, 0)),
          pl.BlockSpec((RB, 1), lambda i: (i, 0)),
          pl.BlockSpec((1, DH), lambda i: (0, 0)),
          pl.BlockSpec((RB, 1), lambda i: (i, 0)),
      ],
      out_specs=pl.BlockSpec((B, DH), lambda i: (0, 0)),
      out_shape=jax.ShapeDtypeStruct((B, DH), jnp.float32),
  )(accp, y, degp_d, bias, batchp)


def _tc_pool(hs, batchp):
  njk = len(hs)

  def body(b_ref, *rest):
    h_refs = rest[:njk]
    out_ref = rest[njk]
    i = pl.program_id(0)

    @pl.when(i == 0)
    def _init():
      out_ref[...] = jnp.full((B, njk * DH), -jnp.inf, jnp.float32)

    rows = i * RB + lax.broadcasted_iota(jnp.int32, (RB, 1), 0)
    valid = rows < N
    bb = b_ref[...]
    hcat = jnp.concatenate([h[...] for h in h_refs], axis=1)
    b_lo = jnp.min(jnp.where(valid, bb, B - 1))
    b_hi = jnp.max(jnp.where(valid, bb, 0))

    def bbody(b, _):
      m = (bb == b) & valid
      mx = jnp.max(jnp.where(m, hcat, -jnp.inf), axis=0, keepdims=True)
      out_ref[pl.ds(b, 1), :] = jnp.maximum(out_ref[pl.ds(b, 1), :], mx)
      return 0
    lax.fori_loop(b_lo, b_hi + 1, bbody, 0)

  return pl.pallas_call(
      body,
      grid=(NBLK,),
      in_specs=[pl.BlockSpec((RB, 1), lambda i: (i, 0))] +
               [pl.BlockSpec((RB, DH), lambda i: (i, 0))] * njk,
      out_specs=pl.BlockSpec((B, njk * DH), lambda i: (0, 0)),
      out_shape=jax.ShapeDtypeStruct((B, njk * DH), jnp.float32),
  )(batchp, *hs)


def _tc_head(pooled, pi, Wg, bg, gg, gb, Wp, bp, pg, pb, Wf, bf, fg, fb):
  def ln(v, g, beta):
    mu = jnp.mean(v, axis=-1, keepdims=True)
    var = jnp.mean((v - mu) ** 2, axis=-1, keepdims=True)
    return (v - mu) / jnp.sqrt(var + 1e-5) * g + beta

  def body(pool_ref, pi_ref, wg_ref, bg_ref, gg_ref, gb_ref,
           wp_ref, bp_ref, pg_ref, pb_ref, wf_ref, bf_ref, fg_ref, fb_ref,
           out_ref):
    g = jnp.dot(pool_ref[...], wg_ref[...],
                preferred_element_type=jnp.float32) + bg_ref[...]
    g = jnp.maximum(ln(g, gg_ref[...], gb_ref[...]), 0.0)
    p = jnp.dot(pi_ref[...], wp_ref[...],
                preferred_element_type=jnp.float32) + bp_ref[...]
    p = jnp.maximum(ln(p, pg_ref[...], pb_ref[...]), 0.0)
    h = jnp.concatenate([g, p], axis=1)
    o = jnp.dot(h, wf_ref[...], preferred_element_type=jnp.float32) \
        + bf_ref[...]
    out_ref[...] = ln(o, fg_ref[...], fb_ref[...])

  args = (pooled, pi, Wg, bg, gg, gb, Wp, bp, pg, pb, Wf, bf, fg, fb)

  def _spec(a):
    zeros = tuple(0 for _ in a.shape)
    return pl.BlockSpec(a.shape, lambda z=zeros: z)

  return pl.pallas_call(
      body,
      in_specs=[_spec(a) for a in args],
      out_specs=pl.BlockSpec((B, Wf.shape[1]), lambda: (0, 0)),
      out_shape=jax.ShapeDtypeStruct((B, Wf.shape[1]), jnp.float32),
  )(*args)


def kernel(x, edge_index, edge_type, batch, pi, W0, b0, Wh, bh, Wg, bg,
           g_gamma, g_beta, Wp, bp, p_gamma, p_beta, Wf, bf, f_gamma,
           f_beta):
  src = edge_index[0].astype(jnp.int32)
  dst = edge_index[1].astype(jnp.int32)
  et0 = edge_type[:, 0].astype(jnp.int32)
  et1 = edge_type[:, 1].astype(jnp.int32)
  xp = jnp.pad(x, ((0, NPAD - N), (0, 0)))
  batchp = jnp.pad(batch.astype(jnp.int32), (0, NPAD - N)).reshape(NPAD, 1)

  srcc, dstc, counts, degp = _sc_prep(src, dst, et0, et1)

  pps = []
  for d in range(NDIM):
    dg = jnp.sum(degp[d], axis=0).reshape(NPAD, 1)
    y = _tc_first(xp, W0[d], dg)
    for l in range(NL):
      accp = _sc_scatter(d, y, srcc, dstc, counts)
      bias = (b0[d] if l == 0 else bh[d, l - 1]).reshape(1, DH)
      if l < NL - 1:
        y, pp = _tc_mid(accp, y, dg, bias, Wh[d, l], batchp)
      else:
        pp = _tc_last(accp, y, dg, bias, batchp)
      pps.append(pp)

  pooled = jnp.concatenate(pps, axis=1)
  out = _tc_head(pooled, pi,
                 Wg, bg.reshape(1, -1), g_gamma.reshape(1, -1),
                 g_beta.reshape(1, -1),
                 Wp, bp.reshape(1, -1), p_gamma.reshape(1, -1),
                 p_beta.reshape(1, -1),
                 Wf, bf.reshape(1, -1), f_gamma.reshape(1, -1),
                 f_beta.reshape(1, -1))
  return out
